# baseline (speedup 1.0000x reference)
import jax
import jax.numpy as jnp
from jax import lax
from jax.experimental import pallas as pl
from jax.experimental.pallas import tpu as pltpu

_ETA = 0.1


def _stage1_body(oh_ref, wstack_ref, b_ref, ginit_ref,
                 ew1_ref, eb1_ref, ew2_ref, eb2_ref, obs_hbm,
                 g_hbm, p_hbm,
                 g_buf, obs_buf, p_buf, sem_in, sem_out, sem_g):
    s, bfull, h = g_buf.shape
    a = oh_ref.shape[2]
    rows = obs_buf.shape[1]
    nchunks = (s * bfull) // rows

    def get(i, slot):
        return pltpu.make_async_copy(
            obs_hbm.at[pl.ds(i * rows, rows), :], obs_buf.at[slot],
            sem_in.at[slot])

    def enc_chunk(i, slot, nxt):
        @pl.when(i + 1 < nchunks)
        def _():
            get(i + 1, nxt).start()

        get(i, slot).wait()
        x = obs_buf[slot]
        hh = jnp.maximum(
            jnp.dot(x, ew1_ref[...], preferred_element_type=jnp.float32)
            + eb1_ref[...], 0.0)
        p = (jnp.dot(hh, ew2_ref[...], preferred_element_type=jnp.float32)
             + eb2_ref[...])

        @pl.when(i >= 2)
        def _():
            pltpu.make_async_copy(
                p_buf.at[slot], p_buf.at[slot], sem_out.at[slot]).wait()

        p_buf[slot] = p.astype(jnp.bfloat16)
        pltpu.make_async_copy(
            p_buf.at[slot], p_hbm.at[pl.ds(i * rows, rows), :],
            sem_out.at[slot]).start()

    g0 = jnp.broadcast_to(ginit_ref[...], (bfull, h))
    g_buf[0:1] = g0.astype(jnp.bfloat16)[None]
    get(0, 0).start()

    def one(t, g):
        oh = oh_ref[t - 1]
        gb = g.astype(jnp.bfloat16)
        gext = jnp.concatenate(
            [oh[:, i:i + 1] * gb for i in range(a)], axis=1)
        z = jnp.dot(gext, wstack_ref[...],
                    preferred_element_type=jnp.float32)
        g2 = jnp.tanh(z + b_ref[...])
        g_buf[pl.ds(t, 1)] = g2.astype(jnp.bfloat16)[None]
        return g2

    def step(i, g):
        t = 1 + i * 4
        for u in range(4):
            g = one(t + u, g)
        enc_chunk(i, lax.rem(i, 2), lax.rem(i + 1, 2))
        return g

    n_unroll = (s - 4) // 4
    g_last = lax.fori_loop(0, n_unroll, step, g0)
    for t in range(s - 3, s):
        g_last = one(t, g_last)
    for i in range(n_unroll, nchunks):
        enc_chunk(i, i % 2, (i + 1) % 2)
    for slot in range(2):
        pltpu.make_async_copy(
            p_buf.at[slot], p_buf.at[slot], sem_out.at[slot]).wait()

    cp = pltpu.make_async_copy(g_buf, g_hbm, sem_g)
    cp.start()
    cp.wait()


def _stage2_body(p_ref, g_ref, dw1_ref, db1_ref, dw2_ref, db2_ref, out_ref):
    nrows = p_ref.shape[0]
    s = g_ref.shape[0]
    nb = g_ref.shape[1]
    chunk = 256

    it = lax.broadcasted_iota(jnp.int32, (s, s), 0)
    ik = lax.broadcasted_iota(jnp.int32, (s, s), 1)
    wmask = jnp.where(ik < it, _ETA, 0.0)
    sel0 = (it + ik) == 0
    ps_parts = []
    for j in range(nb):
        gj = g_ref[:, j, :]
        pj = p_ref[j * s:(j + 1) * s]
        sc = lax.dot_general(gj, gj, (((1,), (1,)), ((), ())),
                             preferred_element_type=jnp.float32)
        scm = jnp.where(sel0, 1.0, sc * wmask)
        ps_parts.append(
            jnp.dot(scm.astype(jnp.bfloat16), pj,
                    preferred_element_type=jnp.float32))

    per = chunk // s
    for c in range(0, nb, per):
        pseq = jnp.concatenate(ps_parts[c:c + per], axis=0)
        h2 = jnp.maximum(
            jnp.dot(pseq, dw1_ref[...], preferred_element_type=jnp.float32)
            + db1_ref[...], 0.0)
        out_ref[c * s:(c + per) * s] = (
            jnp.dot(h2, dw2_ref[...], preferred_element_type=jnp.float32)
            + db2_ref[...])


def kernel(observations, actions, W_trans, b_trans, g_init,
           enc_w1, enc_b1, enc_w2, enc_b2,
           dec_w1, dec_b1, dec_w2, dec_b2):
    b, s, d = observations.shape
    h = g_init.shape[0]
    a = W_trans.shape[0]
    h2 = enc_w1.shape[1]

    obs2 = observations.reshape(b * s, d)
    oh_sb = jnp.transpose(
        jax.nn.one_hot(actions, a, dtype=jnp.bfloat16), (1, 0, 2))
    wstack = jnp.transpose(W_trans, (0, 2, 1)).reshape(
        a * h, h).astype(jnp.bfloat16)

    enc_rows = 256
    g_seq, p_all = pl.pallas_call(
        _stage1_body,
        grid=(1,),
        in_specs=[
            pl.BlockSpec((s, b, a), lambda i: (0, 0, 0)),
            pl.BlockSpec((a * h, h), lambda i: (0, 0)),
            pl.BlockSpec((1, h), lambda i: (0, 0)),
            pl.BlockSpec((1, h), lambda i: (0, 0)),
            pl.BlockSpec((d, h2), lambda i: (0, 0)),
            pl.BlockSpec((1, h2), lambda i: (0, 0)),
            pl.BlockSpec((h2, h), lambda i: (0, 0)),
            pl.BlockSpec((1, h), lambda i: (0, 0)),
            pl.BlockSpec(memory_space=pl.ANY),
        ],
        out_specs=[
            pl.BlockSpec(memory_space=pl.ANY),
            pl.BlockSpec(memory_space=pl.ANY),
        ],
        out_shape=[
            jax.ShapeDtypeStruct((s, b, h), jnp.bfloat16),
            jax.ShapeDtypeStruct((b * s, h), jnp.bfloat16),
        ],
        scratch_shapes=[
            pltpu.VMEM((s, b, h), jnp.bfloat16),
            pltpu.VMEM((2, enc_rows, d), jnp.float32),
            pltpu.VMEM((2, enc_rows, h), jnp.bfloat16),
            pltpu.SemaphoreType.DMA((2,)),
            pltpu.SemaphoreType.DMA((2,)),
            pltpu.SemaphoreType.DMA,
        ],
        compiler_params=pltpu.CompilerParams(
            dimension_semantics=("parallel",)),
    )(oh_sb, wstack, b_trans.reshape(1, h), g_init.reshape(1, h),
      enc_w1, enc_b1.reshape(1, h2), enc_w2, enc_b2.reshape(1, h), obs2)

    blk_b = 8
    rows = blk_b * s
    out2 = pl.pallas_call(
        _stage2_body,
        grid=(b // blk_b,),
        in_specs=[
            pl.BlockSpec((rows, h), lambda i: (i, 0)),
            pl.BlockSpec((s, blk_b, h), lambda i: (0, i, 0)),
            pl.BlockSpec((h, h2), lambda i: (0, 0)),
            pl.BlockSpec((1, h2), lambda i: (0, 0)),
            pl.BlockSpec((h2, d), lambda i: (0, 0)),
            pl.BlockSpec((1, d), lambda i: (0, 0)),
        ],
        out_specs=pl.BlockSpec((rows, d), lambda i: (i, 0)),
        out_shape=jax.ShapeDtypeStruct((b * s, d), jnp.float32),
        compiler_params=pltpu.CompilerParams(
            dimension_semantics=("parallel",)),
    )(p_all, g_seq, dec_w1, dec_b1.reshape(1, h2), dec_w2,
      dec_b2.reshape(1, d))
    return out2.reshape(b, s, d)


# speedup vs baseline: 1.0134x; 1.0134x over previous
import jax
import jax.numpy as jnp
from jax import lax
from jax.experimental import pallas as pl
from jax.experimental.pallas import tpu as pltpu

_ETA = 0.1


def _stage1_body(oh_ref, wstack_ref, b_ref, ginit_ref,
                 ew1_ref, eb1_ref, ew2_ref, eb2_ref, obs_hbm,
                 g_hbm, p_hbm,
                 g_buf, obs_buf, p_buf, sem_in, sem_out, sem_g):
    s, bfull, h = g_buf.shape
    a = oh_ref.shape[2]
    pid = pl.program_id(0)

    @pl.when(pid == 0)
    def _recurrence():
        g0 = jnp.broadcast_to(ginit_ref[...], (bfull, h))
        g_buf[0:1] = g0.astype(jnp.bfloat16)[None]

        def one(t, g):
            oh = oh_ref[t - 1]
            gb = g.astype(jnp.bfloat16)
            gext = jnp.concatenate(
                [oh[:, i:i + 1] * gb for i in range(a)], axis=1)
            z = jnp.dot(gext, wstack_ref[...],
                        preferred_element_type=jnp.float32)
            g2 = jnp.tanh(z + b_ref[...])
            g_buf[pl.ds(t, 1)] = g2.astype(jnp.bfloat16)[None]
            return g2

        def step(i, g):
            t = 1 + i * 4
            for u in range(4):
                g = one(t + u, g)
            return g

        g_last = lax.fori_loop(0, (s - 4) // 4, step, g0)
        for t in range(s - 3, s):
            g_last = one(t, g_last)

        cp = pltpu.make_async_copy(g_buf, g_hbm, sem_g)
        cp.start()
        cp.wait()

    @pl.when(pid == 1)
    def _encoder():
        rows = obs_buf.shape[1]
        nchunks = (s * bfull) // rows

        def get(i, slot):
            return pltpu.make_async_copy(
                obs_hbm.at[pl.ds(i * rows, rows), :], obs_buf.at[slot],
                sem_in.at[slot])

        get(0, 0).start()

        def body(i, carry):
            slot = lax.rem(i, 2)
            nxt = lax.rem(i + 1, 2)

            @pl.when(i + 1 < nchunks)
            def _():
                get(i + 1, nxt).start()

            get(i, slot).wait()
            x = obs_buf[slot]
            hh = jnp.maximum(
                jnp.dot(x, ew1_ref[...], preferred_element_type=jnp.float32)
                + eb1_ref[...], 0.0)
            p = (jnp.dot(hh, ew2_ref[...], preferred_element_type=jnp.float32)
                 + eb2_ref[...])

            @pl.when(i >= 2)
            def _():
                pltpu.make_async_copy(
                    p_buf.at[slot], p_buf.at[slot], sem_out.at[slot]).wait()

            p_buf[slot] = p.astype(jnp.bfloat16)
            pltpu.make_async_copy(
                p_buf.at[slot], p_hbm.at[pl.ds(i * rows, rows), :],
                sem_out.at[slot]).start()
            return carry

        lax.fori_loop(0, nchunks, body, 0)
        for slot in range(2):
            pltpu.make_async_copy(
                p_buf.at[slot], p_buf.at[slot], sem_out.at[slot]).wait()


def _stage2_body(p_ref, g_ref, dw1_ref, db1_ref, dw2_ref, db2_ref, out_ref):
    nrows = p_ref.shape[0]
    s = g_ref.shape[0]
    nb = g_ref.shape[1]
    chunk = 256

    it = lax.broadcasted_iota(jnp.int32, (s, s), 0)
    ik = lax.broadcasted_iota(jnp.int32, (s, s), 1)
    wmask = jnp.where(ik < it, _ETA, 0.0)
    sel0 = (it + ik) == 0
    ps_parts = []
    for j in range(nb):
        gj = g_ref[:, j, :]
        pj = p_ref[j * s:(j + 1) * s]
        sc = lax.dot_general(gj, gj, (((1,), (1,)), ((), ())),
                             preferred_element_type=jnp.float32)
        scm = jnp.where(sel0, 1.0, sc * wmask)
        ps_parts.append(
            jnp.dot(scm.astype(jnp.bfloat16), pj,
                    preferred_element_type=jnp.float32))

    per = chunk // s
    for c in range(0, nb, per):
        pseq = jnp.concatenate(ps_parts[c:c + per], axis=0)
        h2 = jnp.maximum(
            jnp.dot(pseq, dw1_ref[...], preferred_element_type=jnp.float32)
            + db1_ref[...], 0.0)
        out_ref[c * s:(c + per) * s] = (
            jnp.dot(h2, dw2_ref[...], preferred_element_type=jnp.float32)
            + db2_ref[...])


def kernel(observations, actions, W_trans, b_trans, g_init,
           enc_w1, enc_b1, enc_w2, enc_b2,
           dec_w1, dec_b1, dec_w2, dec_b2):
    b, s, d = observations.shape
    h = g_init.shape[0]
    a = W_trans.shape[0]
    h2 = enc_w1.shape[1]

    obs2 = observations.reshape(b * s, d)
    oh_sb = jnp.transpose(
        jax.nn.one_hot(actions, a, dtype=jnp.bfloat16), (1, 0, 2))
    wstack = jnp.transpose(W_trans, (0, 2, 1)).reshape(
        a * h, h).astype(jnp.bfloat16)

    enc_rows = 512
    g_seq, p_all = pl.pallas_call(
        _stage1_body,
        grid=(2,),
        in_specs=[
            pl.BlockSpec((s, b, a), lambda i: (0, 0, 0)),
            pl.BlockSpec((a * h, h), lambda i: (0, 0)),
            pl.BlockSpec((1, h), lambda i: (0, 0)),
            pl.BlockSpec((1, h), lambda i: (0, 0)),
            pl.BlockSpec((d, h2), lambda i: (0, 0)),
            pl.BlockSpec((1, h2), lambda i: (0, 0)),
            pl.BlockSpec((h2, h), lambda i: (0, 0)),
            pl.BlockSpec((1, h), lambda i: (0, 0)),
            pl.BlockSpec(memory_space=pl.ANY),
        ],
        out_specs=[
            pl.BlockSpec(memory_space=pl.ANY),
            pl.BlockSpec(memory_space=pl.ANY),
        ],
        out_shape=[
            jax.ShapeDtypeStruct((s, b, h), jnp.bfloat16),
            jax.ShapeDtypeStruct((b * s, h), jnp.bfloat16),
        ],
        scratch_shapes=[
            pltpu.VMEM((s, b, h), jnp.bfloat16),
            pltpu.VMEM((2, enc_rows, d), jnp.float32),
            pltpu.VMEM((2, enc_rows, h), jnp.bfloat16),
            pltpu.SemaphoreType.DMA((2,)),
            pltpu.SemaphoreType.DMA((2,)),
            pltpu.SemaphoreType.DMA,
        ],
        compiler_params=pltpu.CompilerParams(
            dimension_semantics=("parallel",)),
    )(oh_sb, wstack, b_trans.reshape(1, h), g_init.reshape(1, h),
      enc_w1, enc_b1.reshape(1, h2), enc_w2, enc_b2.reshape(1, h), obs2)

    blk_b = 16
    rows = blk_b * s
    out2 = pl.pallas_call(
        _stage2_body,
        grid=(b // blk_b,),
        in_specs=[
            pl.BlockSpec((rows, h), lambda i: (i, 0)),
            pl.BlockSpec((s, blk_b, h), lambda i: (0, i, 0)),
            pl.BlockSpec((h, h2), lambda i: (0, 0)),
            pl.BlockSpec((1, h2), lambda i: (0, 0)),
            pl.BlockSpec((h2, d), lambda i: (0, 0)),
            pl.BlockSpec((1, d), lambda i: (0, 0)),
        ],
        out_specs=pl.BlockSpec((rows, d), lambda i: (i, 0)),
        out_shape=jax.ShapeDtypeStruct((b * s, d), jnp.float32),
        compiler_params=pltpu.CompilerParams(
            dimension_semantics=("parallel",)),
    )(p_all, g_seq, dec_w1, dec_b1.reshape(1, h2), dec_w2,
      dec_b2.reshape(1, d))
    return out2.reshape(b, s, d)


# speedup vs baseline: 1.0197x; 1.0061x over previous
import jax
import jax.numpy as jnp
from jax import lax
from jax.experimental import pallas as pl
from jax.experimental.pallas import tpu as pltpu

_ETA = 0.1


def _stage1_body(oh_ref, wstack_ref, b_ref, ginit_ref,
                 ew1_ref, eb1_ref, ew2_ref, eb2_ref, obs_hbm,
                 g_hbm, p_hbm,
                 g_buf, obs_buf, p_buf, sem_in, sem_out, sem_g):
    s, bfull, h = g_buf.shape
    a = oh_ref.shape[2]
    pid = pl.program_id(0)

    @pl.when(pid == 0)
    def _recurrence():
        g0 = jnp.broadcast_to(ginit_ref[...], (bfull, h))
        g_buf[0:1] = g0.astype(jnp.bfloat16)[None]

        def one(t, g):
            oh = oh_ref[t - 1]
            gb = g.astype(jnp.bfloat16)
            gext = jnp.concatenate(
                [oh[:, i:i + 1] * gb for i in range(a)], axis=1)
            z = jnp.dot(gext, wstack_ref[...],
                        preferred_element_type=jnp.float32)
            g2 = jnp.tanh(z + b_ref[...])
            g_buf[pl.ds(t, 1)] = g2.astype(jnp.bfloat16)[None]
            return g2

        def step(i, g):
            t = 1 + i * 8
            for u in range(8):
                g = one(t + u, g)
            return g

        g_last = lax.fori_loop(0, (s - 8) // 8, step, g0)
        for t in range(s - 7, s):
            g_last = one(t, g_last)

        cp = pltpu.make_async_copy(g_buf, g_hbm, sem_g)
        cp.start()
        cp.wait()

    @pl.when(pid == 1)
    def _encoder():
        rows = obs_buf.shape[1]
        nchunks = (s * bfull) // rows

        def get(i, slot):
            return pltpu.make_async_copy(
                obs_hbm.at[pl.ds(i * rows, rows), :], obs_buf.at[slot],
                sem_in.at[slot])

        get(0, 0).start()

        def body(i, carry):
            slot = lax.rem(i, 2)
            nxt = lax.rem(i + 1, 2)

            @pl.when(i + 1 < nchunks)
            def _():
                get(i + 1, nxt).start()

            get(i, slot).wait()
            x = obs_buf[slot]
            hh = jnp.maximum(
                jnp.dot(x, ew1_ref[...], preferred_element_type=jnp.float32)
                + eb1_ref[...], 0.0)
            p = (jnp.dot(hh, ew2_ref[...], preferred_element_type=jnp.float32)
                 + eb2_ref[...])

            @pl.when(i >= 2)
            def _():
                pltpu.make_async_copy(
                    p_buf.at[slot], p_buf.at[slot], sem_out.at[slot]).wait()

            p_buf[slot] = p.astype(jnp.bfloat16)
            pltpu.make_async_copy(
                p_buf.at[slot], p_hbm.at[pl.ds(i * rows, rows), :],
                sem_out.at[slot]).start()
            return carry

        lax.fori_loop(0, nchunks, body, 0)
        for slot in range(2):
            pltpu.make_async_copy(
                p_buf.at[slot], p_buf.at[slot], sem_out.at[slot]).wait()


def _stage2_body(p_ref, g_ref, dw1_ref, db1_ref, dw2_ref, db2_ref, out_ref):
    nrows = p_ref.shape[0]
    s = g_ref.shape[0]
    nb = g_ref.shape[1]
    chunk = 256

    it = lax.broadcasted_iota(jnp.int32, (s, s), 0)
    ik = lax.broadcasted_iota(jnp.int32, (s, s), 1)
    wmask = jnp.where(ik < it, _ETA, 0.0)
    sel0 = (it + ik) == 0
    ps_parts = []
    for j in range(nb):
        gj = g_ref[:, j, :]
        pj = p_ref[j * s:(j + 1) * s]
        sc = lax.dot_general(gj, gj, (((1,), (1,)), ((), ())),
                             preferred_element_type=jnp.float32)
        scm = jnp.where(sel0, 1.0, sc * wmask)
        ps_parts.append(
            jnp.dot(scm.astype(jnp.bfloat16), pj,
                    preferred_element_type=jnp.float32))

    per = chunk // s
    for c in range(0, nb, per):
        pseq = jnp.concatenate(ps_parts[c:c + per], axis=0)
        h2 = jnp.maximum(
            jnp.dot(pseq, dw1_ref[...], preferred_element_type=jnp.float32)
            + db1_ref[...], 0.0)
        out_ref[c * s:(c + per) * s] = (
            jnp.dot(h2, dw2_ref[...], preferred_element_type=jnp.float32)
            + db2_ref[...])


def kernel(observations, actions, W_trans, b_trans, g_init,
           enc_w1, enc_b1, enc_w2, enc_b2,
           dec_w1, dec_b1, dec_w2, dec_b2):
    b, s, d = observations.shape
    h = g_init.shape[0]
    a = W_trans.shape[0]
    h2 = enc_w1.shape[1]

    obs2 = observations.reshape(b * s, d)
    oh_sb = jnp.transpose(
        jax.nn.one_hot(actions, a, dtype=jnp.bfloat16), (1, 0, 2))
    wstack = jnp.transpose(W_trans, (0, 2, 1)).reshape(
        a * h, h).astype(jnp.bfloat16)

    enc_rows = 512
    g_seq, p_all = pl.pallas_call(
        _stage1_body,
        grid=(2,),
        in_specs=[
            pl.BlockSpec((s, b, a), lambda i: (0, 0, 0)),
            pl.BlockSpec((a * h, h), lambda i: (0, 0)),
            pl.BlockSpec((1, h), lambda i: (0, 0)),
            pl.BlockSpec((1, h), lambda i: (0, 0)),
            pl.BlockSpec((d, h2), lambda i: (0, 0)),
            pl.BlockSpec((1, h2), lambda i: (0, 0)),
            pl.BlockSpec((h2, h), lambda i: (0, 0)),
            pl.BlockSpec((1, h), lambda i: (0, 0)),
            pl.BlockSpec(memory_space=pl.ANY),
        ],
        out_specs=[
            pl.BlockSpec(memory_space=pl.ANY),
            pl.BlockSpec(memory_space=pl.ANY),
        ],
        out_shape=[
            jax.ShapeDtypeStruct((s, b, h), jnp.bfloat16),
            jax.ShapeDtypeStruct((b * s, h), jnp.bfloat16),
        ],
        scratch_shapes=[
            pltpu.VMEM((s, b, h), jnp.bfloat16),
            pltpu.VMEM((2, enc_rows, d), jnp.float32),
            pltpu.VMEM((2, enc_rows, h), jnp.bfloat16),
            pltpu.SemaphoreType.DMA((2,)),
            pltpu.SemaphoreType.DMA((2,)),
            pltpu.SemaphoreType.DMA,
        ],
        compiler_params=pltpu.CompilerParams(
            dimension_semantics=("parallel",)),
    )(oh_sb, wstack, b_trans.reshape(1, h), g_init.reshape(1, h),
      enc_w1, enc_b1.reshape(1, h2), enc_w2, enc_b2.reshape(1, h), obs2)

    blk_b = 16
    rows = blk_b * s
    out2 = pl.pallas_call(
        _stage2_body,
        grid=(b // blk_b,),
        in_specs=[
            pl.BlockSpec((rows, h), lambda i: (i, 0)),
            pl.BlockSpec((s, blk_b, h), lambda i: (0, i, 0)),
            pl.BlockSpec((h, h2), lambda i: (0, 0)),
            pl.BlockSpec((1, h2), lambda i: (0, 0)),
            pl.BlockSpec((h2, d), lambda i: (0, 0)),
            pl.BlockSpec((1, d), lambda i: (0, 0)),
        ],
        out_specs=pl.BlockSpec((rows, d), lambda i: (i, 0)),
        out_shape=jax.ShapeDtypeStruct((b * s, d), jnp.float32),
        compiler_params=pltpu.CompilerParams(
            dimension_semantics=("parallel",)),
    )(p_all, g_seq, dec_w1, dec_b1.reshape(1, h2), dec_w2,
      dec_b2.reshape(1, d))
    return out2.reshape(b, s, d)
